# Initial kernel scaffold; baseline (speedup 1.0000x reference)
import jax
import jax.numpy as jnp
from jax.experimental import pallas as pl


def kernel(atom_state, messages, connectivity, W, U, b):
    raise NotImplementedError("write your pallas kernel here")


# speedup vs baseline: 7.9103x; 7.9103x over previous
import functools

import jax
import jax.numpy as jnp
from jax.experimental import pallas as pl

ATOM_DIM = 256
B, N, E = 256, 128, 256
BB = 8


def _gru_tc_kernel(idx_ref, msg_ref, h_ref, w_ref, u_ref, b_ref, out_ref):
    idx = idx_ref[:, 0, :]
    node_ids = jax.lax.broadcasted_iota(jnp.int32, (BB, N, E), 1)
    onehot = (idx[:, None, :] == node_ids).astype(jnp.float32)
    agg = jax.lax.dot_general(
        onehot, msg_ref[...],
        dimension_numbers=(((2,), (1,)), ((0,), (0,))),
        preferred_element_type=jnp.float32,
    )

    x = agg.reshape(BB * N, ATOM_DIM)
    h = h_ref[...].reshape(BB * N, ATOM_DIM)
    mx = jnp.dot(x, w_ref[...], preferred_element_type=jnp.float32) + b_ref[0]
    mh = jnp.dot(h, u_ref[...], preferred_element_type=jnp.float32) + b_ref[1]
    xz, xr, xh = mx[:, :ATOM_DIM], mx[:, ATOM_DIM:2 * ATOM_DIM], mx[:, 2 * ATOM_DIM:]
    hz, hr, hh_ = mh[:, :ATOM_DIM], mh[:, ATOM_DIM:2 * ATOM_DIM], mh[:, 2 * ATOM_DIM:]
    z = jax.nn.sigmoid(xz + hz)
    r = jax.nn.sigmoid(xr + hr)
    hh = jnp.tanh(xh + r * hh_)
    out_ref[...] = (z * h + (1.0 - z) * hh).reshape(BB, N, ATOM_DIM)


@functools.partial(jax.jit, static_argnames=())
def kernel(atom_state, messages, connectivity, W, U, b):
    tgt_idx = connectivity[:, :, 1].astype(jnp.int32).reshape(B, 1, E)
    grid = (B // BB,)
    return pl.pallas_call(
        _gru_tc_kernel,
        grid=grid,
        in_specs=[
            pl.BlockSpec((BB, 1, E), lambda i: (i, 0, 0)),
            pl.BlockSpec((BB, E, ATOM_DIM), lambda i: (i, 0, 0)),
            pl.BlockSpec((BB, N, ATOM_DIM), lambda i: (i, 0, 0)),
            pl.BlockSpec((ATOM_DIM, 3 * ATOM_DIM), lambda i: (0, 0)),
            pl.BlockSpec((ATOM_DIM, 3 * ATOM_DIM), lambda i: (0, 0)),
            pl.BlockSpec((2, 3 * ATOM_DIM), lambda i: (0, 0)),
        ],
        out_specs=pl.BlockSpec((BB, N, ATOM_DIM), lambda i: (i, 0, 0)),
        out_shape=jax.ShapeDtypeStruct((B, N, ATOM_DIM), jnp.float32),
    )(tgt_idx, messages, atom_state, W, U, b)


# speedup vs baseline: 7.9179x; 1.0010x over previous
import functools

import jax
import jax.numpy as jnp
from jax.experimental import pallas as pl

ATOM_DIM = 256
B, N, E = 256, 128, 256
BB = 8


def _gru_tc_kernel(idx_ref, msg_ref, h_ref, w_ref, u_ref, b_ref, out_ref):
    idx = idx_ref[:, 0, :]
    node_ids = jax.lax.broadcasted_iota(jnp.int32, (BB, N, E), 1)
    onehot = (idx[:, None, :] == node_ids).astype(jnp.float32)
    agg = jax.lax.dot_general(
        onehot, msg_ref[...],
        dimension_numbers=(((2,), (1,)), ((0,), (0,))),
        preferred_element_type=jnp.float32,
    )

    x = agg.reshape(BB * N, ATOM_DIM).astype(jnp.bfloat16)
    h = h_ref[...].reshape(BB * N, ATOM_DIM)
    hb = h.astype(jnp.bfloat16)
    mx = jnp.dot(x, w_ref[...].astype(jnp.bfloat16),
                 preferred_element_type=jnp.float32) + b_ref[0]
    mh = jnp.dot(hb, u_ref[...].astype(jnp.bfloat16),
                 preferred_element_type=jnp.float32) + b_ref[1]
    xz, xr, xh = mx[:, :ATOM_DIM], mx[:, ATOM_DIM:2 * ATOM_DIM], mx[:, 2 * ATOM_DIM:]
    hz, hr, hh_ = mh[:, :ATOM_DIM], mh[:, ATOM_DIM:2 * ATOM_DIM], mh[:, 2 * ATOM_DIM:]
    z = jax.nn.sigmoid(xz + hz)
    r = jax.nn.sigmoid(xr + hr)
    hh = jnp.tanh(xh + r * hh_)
    out_ref[...] = (z * h + (1.0 - z) * hh).reshape(BB, N, ATOM_DIM)


@functools.partial(jax.jit, static_argnames=())
def kernel(atom_state, messages, connectivity, W, U, b):
    tgt_idx = connectivity[:, :, 1].astype(jnp.int32).reshape(B, 1, E)
    grid = (B // BB,)
    return pl.pallas_call(
        _gru_tc_kernel,
        grid=grid,
        in_specs=[
            pl.BlockSpec((BB, 1, E), lambda i: (i, 0, 0)),
            pl.BlockSpec((BB, E, ATOM_DIM), lambda i: (i, 0, 0)),
            pl.BlockSpec((BB, N, ATOM_DIM), lambda i: (i, 0, 0)),
            pl.BlockSpec((ATOM_DIM, 3 * ATOM_DIM), lambda i: (0, 0)),
            pl.BlockSpec((ATOM_DIM, 3 * ATOM_DIM), lambda i: (0, 0)),
            pl.BlockSpec((2, 3 * ATOM_DIM), lambda i: (0, 0)),
        ],
        out_specs=pl.BlockSpec((BB, N, ATOM_DIM), lambda i: (i, 0, 0)),
        out_shape=jax.ShapeDtypeStruct((B, N, ATOM_DIM), jnp.float32),
    )(tgt_idx, messages, atom_state, W, U, b)


# speedup vs baseline: 9.0614x; 1.1444x over previous
import functools

import jax
import jax.numpy as jnp
from jax.experimental import pallas as pl

ATOM_DIM = 256
B, N, E = 256, 128, 256
BB = 16


def _gru_tc_kernel(idx_ref, msg_ref, h_ref, w_ref, u_ref, b_ref, out_ref):
    idx = idx_ref[:, 0, :]
    node_ids = jax.lax.broadcasted_iota(jnp.int32, (BB, N, E), 1)
    onehot = (idx[:, None, :] == node_ids).astype(jnp.float32)
    agg = jax.lax.dot_general(
        onehot, msg_ref[...],
        dimension_numbers=(((2,), (1,)), ((0,), (0,))),
        preferred_element_type=jnp.float32,
    )

    x = agg.reshape(BB * N, ATOM_DIM).astype(jnp.bfloat16)
    h = h_ref[...].reshape(BB * N, ATOM_DIM)
    hb = h.astype(jnp.bfloat16)
    mx = jnp.dot(x, w_ref[...].astype(jnp.bfloat16),
                 preferred_element_type=jnp.float32) + b_ref[0]
    mh = jnp.dot(hb, u_ref[...].astype(jnp.bfloat16),
                 preferred_element_type=jnp.float32) + b_ref[1]
    xz, xr, xh = mx[:, :ATOM_DIM], mx[:, ATOM_DIM:2 * ATOM_DIM], mx[:, 2 * ATOM_DIM:]
    hz, hr, hh_ = mh[:, :ATOM_DIM], mh[:, ATOM_DIM:2 * ATOM_DIM], mh[:, 2 * ATOM_DIM:]
    z = jax.nn.sigmoid(xz + hz)
    r = jax.nn.sigmoid(xr + hr)
    hh = jnp.tanh(xh + r * hh_)
    out_ref[...] = (z * h + (1.0 - z) * hh).reshape(BB, N, ATOM_DIM)


@functools.partial(jax.jit, static_argnames=())
def kernel(atom_state, messages, connectivity, W, U, b):
    tgt_idx = connectivity[:, :, 1].astype(jnp.int32).reshape(B, 1, E)
    grid = (B // BB,)
    return pl.pallas_call(
        _gru_tc_kernel,
        grid=grid,
        in_specs=[
            pl.BlockSpec((BB, 1, E), lambda i: (i, 0, 0)),
            pl.BlockSpec((BB, E, ATOM_DIM), lambda i: (i, 0, 0)),
            pl.BlockSpec((BB, N, ATOM_DIM), lambda i: (i, 0, 0)),
            pl.BlockSpec((ATOM_DIM, 3 * ATOM_DIM), lambda i: (0, 0)),
            pl.BlockSpec((ATOM_DIM, 3 * ATOM_DIM), lambda i: (0, 0)),
            pl.BlockSpec((2, 3 * ATOM_DIM), lambda i: (0, 0)),
        ],
        out_specs=pl.BlockSpec((BB, N, ATOM_DIM), lambda i: (i, 0, 0)),
        out_shape=jax.ShapeDtypeStruct((B, N, ATOM_DIM), jnp.float32),
    )(tgt_idx, messages, atom_state, W, U, b)


# speedup vs baseline: 9.3027x; 1.0266x over previous
import functools

import jax
import jax.numpy as jnp
from jax.experimental import pallas as pl

ATOM_DIM = 256
B, N, E = 256, 128, 256
BB = 32


def _gru_tc_kernel(idx_ref, msg_ref, h_ref, w_ref, u_ref, b_ref, out_ref):
    idx = idx_ref[:, 0, :]
    node_ids = jax.lax.broadcasted_iota(jnp.int32, (BB, N, E), 1)
    onehot = (idx[:, None, :] == node_ids).astype(jnp.float32)
    agg = jax.lax.dot_general(
        onehot, msg_ref[...],
        dimension_numbers=(((2,), (1,)), ((0,), (0,))),
        preferred_element_type=jnp.float32,
    )

    x = agg.reshape(BB * N, ATOM_DIM).astype(jnp.bfloat16)
    h = h_ref[...].reshape(BB * N, ATOM_DIM)
    hb = h.astype(jnp.bfloat16)
    mx = jnp.dot(x, w_ref[...].astype(jnp.bfloat16),
                 preferred_element_type=jnp.float32) + b_ref[0]
    mh = jnp.dot(hb, u_ref[...].astype(jnp.bfloat16),
                 preferred_element_type=jnp.float32) + b_ref[1]
    xz, xr, xh = mx[:, :ATOM_DIM], mx[:, ATOM_DIM:2 * ATOM_DIM], mx[:, 2 * ATOM_DIM:]
    hz, hr, hh_ = mh[:, :ATOM_DIM], mh[:, ATOM_DIM:2 * ATOM_DIM], mh[:, 2 * ATOM_DIM:]
    z = jax.nn.sigmoid(xz + hz)
    r = jax.nn.sigmoid(xr + hr)
    hh = jnp.tanh(xh + r * hh_)
    out_ref[...] = (z * h + (1.0 - z) * hh).reshape(BB, N, ATOM_DIM)


@functools.partial(jax.jit, static_argnames=())
def kernel(atom_state, messages, connectivity, W, U, b):
    tgt_idx = connectivity[:, :, 1].astype(jnp.int32).reshape(B, 1, E)
    grid = (B // BB,)
    return pl.pallas_call(
        _gru_tc_kernel,
        grid=grid,
        in_specs=[
            pl.BlockSpec((BB, 1, E), lambda i: (i, 0, 0)),
            pl.BlockSpec((BB, E, ATOM_DIM), lambda i: (i, 0, 0)),
            pl.BlockSpec((BB, N, ATOM_DIM), lambda i: (i, 0, 0)),
            pl.BlockSpec((ATOM_DIM, 3 * ATOM_DIM), lambda i: (0, 0)),
            pl.BlockSpec((ATOM_DIM, 3 * ATOM_DIM), lambda i: (0, 0)),
            pl.BlockSpec((2, 3 * ATOM_DIM), lambda i: (0, 0)),
        ],
        out_specs=pl.BlockSpec((BB, N, ATOM_DIM), lambda i: (i, 0, 0)),
        out_shape=jax.ShapeDtypeStruct((B, N, ATOM_DIM), jnp.float32),
    )(tgt_idx, messages, atom_state, W, U, b)
